# baseline (speedup 1.0000x reference)
import functools

import jax
import jax.numpy as jnp
from jax.experimental import pallas as pl


def _writer_body(TM, K, mk_ref, mv_ref, q_ref, v_ref, w_ref, idx_ref,
                 ok_ref, ov_ref):
    t = pl.program_id(0)
    B = q_ref.shape[0]
    D = q_ref.shape[1]

    w = w_ref[...]
    w = jnp.where(w > 0.01, w, 0.0)
    wq = w * q_ref[...]
    wv = w * v_ref[...]
    idx = idx_ref[...]

    row = t * TM + jax.lax.broadcasted_iota(jnp.int32, (TM, 1), 0)

    acc_k = jnp.zeros((TM, D), jnp.float32)
    acc_v = jnp.zeros((TM, D), jnp.float32)
    acc_w = jnp.zeros((TM, 1), jnp.float32)
    for k in range(K):
        oh = (row == idx[k:k + 1, :]).astype(jnp.float32)
        acc_k = acc_k + jnp.dot(oh, wq, preferred_element_type=jnp.float32)
        acc_v = acc_v + jnp.dot(oh, wv, preferred_element_type=jnp.float32)
        acc_w = acc_w + jnp.dot(oh, w, preferred_element_type=jnp.float32)

    valid = acc_w > 0.0
    safe = jnp.where(valid, acc_w, 1.0)
    scale = 0.1
    ok_ref[...] = mk_ref[...] + scale * jnp.where(valid, acc_k / safe, 0.0)
    ov_ref[...] = mv_ref[...] + scale * jnp.where(valid, acc_v / safe, 0.0)


def _build_call(M, D, B, K, TM):
    body = functools.partial(_writer_body, TM, K)
    return pl.pallas_call(
        body,
        grid=(M // TM,),
        in_specs=[
            pl.BlockSpec((TM, D), lambda i: (i, 0)),
            pl.BlockSpec((TM, D), lambda i: (i, 0)),
            pl.BlockSpec((B, D), lambda i: (0, 0)),
            pl.BlockSpec((B, D), lambda i: (0, 0)),
            pl.BlockSpec((B, 1), lambda i: (0, 0)),
            pl.BlockSpec((K, B), lambda i: (0, 0)),
        ],
        out_specs=[
            pl.BlockSpec((TM, D), lambda i: (i, 0)),
            pl.BlockSpec((TM, D), lambda i: (i, 0)),
        ],
        out_shape=[
            jax.ShapeDtypeStruct((M, D), jnp.float32),
            jax.ShapeDtypeStruct((M, D), jnp.float32),
        ],
    )


def kernel(memory_keys, memory_values, write_query, write_value,
           gate_weights, top_indices):
    M, D = memory_keys.shape
    B, K = top_indices.shape
    TM = min(512, M)
    call = _build_call(M, D, B, K, TM)
    ok, ov = call(
        memory_keys,
        memory_values,
        write_query,
        write_value,
        gate_weights.reshape(B, 1).astype(jnp.float32),
        top_indices.astype(jnp.int32).T,
    )
    return (ok, ov)


# speedup vs baseline: 2.9297x; 2.9297x over previous
import functools

import jax
import jax.numpy as jnp
from jax.experimental import pallas as pl


def _writer_body(TM, K, mk_ref, mv_ref, q_ref, v_ref, w_ref, idx_ref,
                 ok_ref, ov_ref):
    t = pl.program_id(0)
    B = q_ref.shape[0]
    D = q_ref.shape[1]

    w = w_ref[...]
    w = jnp.where(w > 0.01, w, 0.0)
    wq = w * q_ref[...]
    wv = w * v_ref[...]
    idx = idx_ref[...]

    row = t * TM + jax.lax.broadcasted_iota(jnp.int32, (TM, 1), 0)

    cnt = jnp.zeros((TM, B), jnp.float32)
    for k in range(K):
        cnt = cnt + (row == idx[k:k + 1, :]).astype(jnp.float32)
    acc_k = jnp.dot(cnt, wq, preferred_element_type=jnp.float32)
    acc_v = jnp.dot(cnt, wv, preferred_element_type=jnp.float32)
    acc_w = jnp.dot(cnt, w, preferred_element_type=jnp.float32)

    valid = acc_w > 0.0
    safe = jnp.where(valid, acc_w, 1.0)
    scale = 0.1
    ok_ref[...] = mk_ref[...] + scale * jnp.where(valid, acc_k / safe, 0.0)
    ov_ref[...] = mv_ref[...] + scale * jnp.where(valid, acc_v / safe, 0.0)


def _build_call(M, D, B, K, TM):
    body = functools.partial(_writer_body, TM, K)
    return pl.pallas_call(
        body,
        grid=(M // TM,),
        in_specs=[
            pl.BlockSpec((TM, D), lambda i: (i, 0)),
            pl.BlockSpec((TM, D), lambda i: (i, 0)),
            pl.BlockSpec((B, D), lambda i: (0, 0)),
            pl.BlockSpec((B, D), lambda i: (0, 0)),
            pl.BlockSpec((B, 1), lambda i: (0, 0)),
            pl.BlockSpec((K, B), lambda i: (0, 0)),
        ],
        out_specs=[
            pl.BlockSpec((TM, D), lambda i: (i, 0)),
            pl.BlockSpec((TM, D), lambda i: (i, 0)),
        ],
        out_shape=[
            jax.ShapeDtypeStruct((M, D), jnp.float32),
            jax.ShapeDtypeStruct((M, D), jnp.float32),
        ],
    )


def kernel(memory_keys, memory_values, write_query, write_value,
           gate_weights, top_indices):
    M, D = memory_keys.shape
    B, K = top_indices.shape
    TM = min(512, M)
    call = _build_call(M, D, B, K, TM)
    ok, ov = call(
        memory_keys,
        memory_values,
        write_query,
        write_value,
        gate_weights.reshape(B, 1).astype(jnp.float32),
        top_indices.astype(jnp.int32).T,
    )
    return (ok, ov)


# speedup vs baseline: 3.6967x; 1.2618x over previous
import functools

import jax
import jax.numpy as jnp
from jax.experimental import pallas as pl


def _writer_body(TM, K, mk_ref, mv_ref, q_ref, v_ref, w_ref, idx_ref,
                 ok_ref, ov_ref):
    t = pl.program_id(0)
    B = q_ref.shape[0]
    D = q_ref.shape[1]

    w = w_ref[...]
    w = jnp.where(w > 0.01, w, 0.0)
    wq = w * q_ref[...]
    wv = w * v_ref[...]
    idx = idx_ref[...]

    row = t * TM + jax.lax.broadcasted_iota(jnp.int32, (TM, 1), 0)

    cnt = jnp.zeros((TM, B), jnp.float32)
    for k in range(K):
        cnt = cnt + (row == idx[k:k + 1, :]).astype(jnp.float32)
    cnt16 = cnt.astype(jnp.bfloat16)
    payload = jnp.concatenate([wq, wv], axis=1).astype(jnp.bfloat16)
    acc_kv = jnp.dot(cnt16, payload, preferred_element_type=jnp.float32)
    acc_k = acc_kv[:, :D]
    acc_v = acc_kv[:, D:]
    acc_w = jnp.dot(cnt, w, preferred_element_type=jnp.float32)

    valid = acc_w > 0.0
    safe = jnp.where(valid, acc_w, 1.0)
    scale = 0.1
    ok_ref[...] = mk_ref[...] + scale * jnp.where(valid, acc_k / safe, 0.0)
    ov_ref[...] = mv_ref[...] + scale * jnp.where(valid, acc_v / safe, 0.0)


def _build_call(M, D, B, K, TM):
    body = functools.partial(_writer_body, TM, K)
    return pl.pallas_call(
        body,
        grid=(M // TM,),
        in_specs=[
            pl.BlockSpec((TM, D), lambda i: (i, 0)),
            pl.BlockSpec((TM, D), lambda i: (i, 0)),
            pl.BlockSpec((B, D), lambda i: (0, 0)),
            pl.BlockSpec((B, D), lambda i: (0, 0)),
            pl.BlockSpec((B, 1), lambda i: (0, 0)),
            pl.BlockSpec((K, B), lambda i: (0, 0)),
        ],
        out_specs=[
            pl.BlockSpec((TM, D), lambda i: (i, 0)),
            pl.BlockSpec((TM, D), lambda i: (i, 0)),
        ],
        out_shape=[
            jax.ShapeDtypeStruct((M, D), jnp.float32),
            jax.ShapeDtypeStruct((M, D), jnp.float32),
        ],
    )


def kernel(memory_keys, memory_values, write_query, write_value,
           gate_weights, top_indices):
    M, D = memory_keys.shape
    B, K = top_indices.shape
    TM = min(512, M)
    call = _build_call(M, D, B, K, TM)
    ok, ov = call(
        memory_keys,
        memory_values,
        write_query,
        write_value,
        gate_weights.reshape(B, 1).astype(jnp.float32),
        top_indices.astype(jnp.int32).T,
    )
    return (ok, ov)


# speedup vs baseline: 3.7289x; 1.0087x over previous
import functools

import jax
import jax.numpy as jnp
from jax.experimental import pallas as pl


def _writer_body(TM, K, mk_ref, mv_ref, q_ref, v_ref, w_ref, idx_ref,
                 ok_ref, ov_ref):
    t = pl.program_id(0)
    B = q_ref.shape[0]
    D = q_ref.shape[1]

    w = w_ref[...]
    w = jnp.where(w > 0.01, w, 0.0)
    wq = w * q_ref[...]
    wv = w * v_ref[...]
    idx = idx_ref[...]

    row = t * TM + jax.lax.broadcasted_iota(jnp.int32, (TM, 1), 0)

    cnt = jnp.zeros((TM, B), jnp.float32)
    for k in range(K):
        cnt = cnt + (row == idx[k:k + 1, :]).astype(jnp.float32)
    cnt16 = cnt.astype(jnp.bfloat16)
    w_pad = jnp.pad(w, ((0, 0), (0, D - 1)))
    payload = jnp.concatenate([wq, wv, w_pad], axis=1).astype(jnp.bfloat16)
    acc = jnp.dot(cnt16, payload, preferred_element_type=jnp.float32)
    acc_k = acc[:, :D]
    acc_v = acc[:, D:2 * D]
    acc_w = acc[:, 2 * D:2 * D + 1]

    valid = acc_w > 0.0
    safe = jnp.where(valid, acc_w, 1.0)
    scale = 0.1
    ok_ref[...] = mk_ref[...] + scale * jnp.where(valid, acc_k / safe, 0.0)
    ov_ref[...] = mv_ref[...] + scale * jnp.where(valid, acc_v / safe, 0.0)


def _build_call(M, D, B, K, TM):
    body = functools.partial(_writer_body, TM, K)
    return pl.pallas_call(
        body,
        grid=(M // TM,),
        in_specs=[
            pl.BlockSpec((TM, D), lambda i: (i, 0)),
            pl.BlockSpec((TM, D), lambda i: (i, 0)),
            pl.BlockSpec((B, D), lambda i: (0, 0)),
            pl.BlockSpec((B, D), lambda i: (0, 0)),
            pl.BlockSpec((B, 1), lambda i: (0, 0)),
            pl.BlockSpec((K, B), lambda i: (0, 0)),
        ],
        out_specs=[
            pl.BlockSpec((TM, D), lambda i: (i, 0)),
            pl.BlockSpec((TM, D), lambda i: (i, 0)),
        ],
        out_shape=[
            jax.ShapeDtypeStruct((M, D), jnp.float32),
            jax.ShapeDtypeStruct((M, D), jnp.float32),
        ],
    )


def kernel(memory_keys, memory_values, write_query, write_value,
           gate_weights, top_indices):
    M, D = memory_keys.shape
    B, K = top_indices.shape
    TM = min(512, M)
    call = _build_call(M, D, B, K, TM)
    ok, ov = call(
        memory_keys,
        memory_values,
        write_query,
        write_value,
        gate_weights.reshape(B, 1).astype(jnp.float32),
        top_indices.astype(jnp.int32).T,
    )
    return (ok, ov)
